# baseline (speedup 1.0000x reference)
import functools

import jax
import jax.numpy as jnp
from jax import lax
from jax.experimental import pallas as pl
from jax.experimental.pallas import tpu as pltpu
from jax.experimental.pallas import tpu_sc as plsc

NC = 2
NS = 16
NW = NC * NS
L = 16

N = 2_000_000
K = 1024
UPT = K // NS

C0 = (N // NW) // L * L
TAIL = N - NW * C0
CT = C0 + TAIL

CH = 8192
_starts = list(range(0, CT, CH))
CHUNKS = [(o, min(CH, CT - o)) for o in _starts]
NCH = len(CHUNKS)
WCHUNKS = [(o, min(CH, C0 - o)) for o in _starts if o < C0]

UNROLL = 8

_mesh = plsc.VectorSubcoreMesh(core_axis_name="c", subcore_axis_name="s")
_sc_params = pltpu.CompilerParams(needs_layout_passes=False)


def _flags_call(labels):
    @functools.partial(
        pl.kernel,
        mesh=_mesh,
        out_type=jax.ShapeDtypeStruct((NW, K), jnp.int32),
        compiler_params=_sc_params,
        scratch_types=[
            pltpu.VMEM((CT,), jnp.int32),
            pltpu.VMEM((K,), jnp.int32),
        ]
        + [pltpu.SemaphoreType.DMA] * NCH,
    )
    def k(labels_hbm, flags_hbm, lab_v, flg_v, *sems):
        c = lax.axis_index("c")
        s = lax.axis_index("s")
        wid = s * NC + c
        base = wid * C0

        def in_copy(j):
            off, sz = CHUNKS[j]
            return pltpu.make_async_copy(
                labels_hbm.at[pl.ds(base + off, sz)],
                lab_v.at[pl.ds(off, sz)],
                sems[j],
            )

        for j in range(NCH):
            in_copy(j).start()

        zeros = jnp.zeros((L,), jnp.int32)
        for i in range(K // L):
            flg_v[pl.ds(i * L, L)] = zeros

        ones = jnp.ones((L,), jnp.int32)
        for j in range(NCH):
            off, sz = CHUNKS[j]
            in_copy(j).wait()

            def body(i, carry, off=off):
                lv = lab_v[pl.ds(off + i * L, L)]
                plsc.store_scatter(flg_v, [lv - 1], ones)
                return carry

            lax.fori_loop(0, sz // L, body, 0, unroll=UNROLL)

        pltpu.sync_copy(flg_v, flags_hbm.at[wid])

    return k(labels)


def _rank_gather_call(labels, flags, peak):
    @functools.partial(
        pl.kernel,
        mesh=_mesh,
        out_type=jax.ShapeDtypeStruct((N,), jnp.int32),
        compiler_params=_sc_params,
        scratch_types=[
            pltpu.VMEM((CT,), jnp.int32),
            pltpu.VMEM((NW, K), jnp.int32),
            pltpu.VMEM((K,), jnp.int32),
            pltpu.VMEM((K,), jnp.float32),
            pltpu.VMEM((K,), jnp.float32),
            pltpu.VMEM((K,), jnp.int32),
            pltpu.VMEM_SHARED((K,), jnp.int32),
        ]
        + [pltpu.SemaphoreType.DMA] * (2 * NCH + 3),
    )
    def k(
        labels_hbm,
        flags_hbm,
        peak_hbm,
        out_hbm,
        lab_v,
        flg_all,
        occ_v,
        peak_v,
        aa_v,
        map_v,
        map_sp,
        *sems,
    ):
        c = lax.axis_index("c")
        s = lax.axis_index("s")
        wid = s * NC + c
        base = wid * C0
        sems_in = sems[:NCH]
        sems_out = sems[NCH : 2 * NCH]
        sem_tail, sem_f, sem_p = sems[2 * NCH :]

        def in_copy(j):
            off, sz = CHUNKS[j]
            return pltpu.make_async_copy(
                labels_hbm.at[pl.ds(base + off, sz)],
                lab_v.at[pl.ds(off, sz)],
                sems_in[j],
            )

        def out_copy(j):
            off, sz = WCHUNKS[j]
            return pltpu.make_async_copy(
                lab_v.at[pl.ds(off, sz)],
                out_hbm.at[pl.ds(base + off, sz)],
                sems_out[j],
            )

        def tail_copy():
            return pltpu.make_async_copy(
                lab_v.at[pl.ds(C0, TAIL)],
                out_hbm.at[pl.ds(base + C0, TAIL)],
                sem_tail,
            )

        for j in range(NCH):
            in_copy(j).start()

        flags_cp = pltpu.make_async_copy(flags_hbm, flg_all, sem_f)
        peak_cp = pltpu.make_async_copy(peak_hbm, peak_v, sem_p)
        flags_cp.start()
        peak_cp.start()
        flags_cp.wait()
        peak_cp.wait()

        iota16 = lax.broadcasted_iota(jnp.int32, (L,), 0)
        zero16 = jnp.zeros((L,), jnp.int32)

        def merge_body(jv, kpart):
            acc = flg_all[0, pl.ds(jv * L, L)]
            for r in range(1, NW):
                acc = acc + flg_all[r, pl.ds(jv * L, L)]
            occ_v[pl.ds(jv * L, L)] = acc
            jidx = iota16 + jv * L
            return jnp.maximum(kpart, jnp.where(acc > 0, jidx + 1, 0))

        kpart = lax.fori_loop(0, K // L, merge_body, zero16)
        kmax = jnp.max(kpart)

        inf = jnp.float32(jnp.inf)

        def aa_body(jv, carry):
            occ = occ_v[pl.ds(jv * L, L)]
            jidx = iota16 + jv * L
            empty_in_range = jnp.logical_and(occ == 0, jidx < kmax)
            aa_v[pl.ds(jv * L, L)] = jnp.where(
                empty_in_range, inf, peak_v[pl.ds(jv * L, L)]
            )
            return carry

        lax.fori_loop(0, K // L, aa_body, 0)

        u0 = s * UPT
        lane0 = iota16 == 0

        def rank_body(i, carry):
            unit = u0 + i
            vi = zero16 + unit
            av = plsc.load_gather(aa_v, [vi])
            acc = zero16
            for j in range(K // L):
                ajv = aa_v[pl.ds(j * L, L)]
                jidx = iota16 + j * L
                pred = jnp.logical_or(
                    ajv < av,
                    jnp.logical_and(ajv == av, jidx < unit),
                )
                acc = acc + jnp.where(pred, 1, 0)
            rank = jnp.sum(acc) + 1
            plsc.store_scatter(map_v, [vi], zero16 + rank, mask=lane0)
            return carry

        lax.fori_loop(0, UPT, rank_body, 0)

        pltpu.sync_copy(map_v.at[pl.ds(u0, UPT)], map_sp.at[pl.ds(u0, UPT)])
        plsc.subcore_barrier()
        pltpu.sync_copy(map_sp, map_v)

        for j in range(NCH):
            off, sz = CHUNKS[j]
            in_copy(j).wait()

            def body(i, carry, off=off):
                lv = lab_v[pl.ds(off + i * L, L)]
                lab_v[pl.ds(off + i * L, L)] = plsc.load_gather(
                    map_v, [lv - 1]
                )
                return carry

            lax.fori_loop(0, sz // L, body, 0, unroll=UNROLL)
            out_copy(j).start()

        @pl.when(wid == NW - 1)
        def _():
            tail_copy().start()
            tail_copy().wait()

        for j in range(NCH):
            out_copy(j).wait()

    return k(labels, flags, peak)


def kernel(labels, peak_channel_indices):
    flags = _flags_call(labels)
    return _rank_gather_call(labels, flags, peak_channel_indices)


# speedup vs baseline: 1.2603x; 1.2603x over previous
import functools

import jax
import jax.numpy as jnp
from jax import lax
from jax.experimental import pallas as pl
from jax.experimental.pallas import tpu as pltpu
from jax.experimental.pallas import tpu_sc as plsc

NC = 2
NS = 16
NW = NC * NS
L = 16

N = 2_000_000
K = 1024

C0 = (N // NW) // L * L
TAIL = N - NW * C0
CT = C0 + TAIL

CH = 8192
_starts = list(range(0, CT, CH))
CHUNKS = [(o, min(CH, CT - o)) for o in _starts]
NCH = len(CHUNKS)
WCHUNKS = [(o, min(CH, C0 - o)) for o in _starts if o < C0]

UNROLL = 8

_mesh = plsc.VectorSubcoreMesh(core_axis_name="c", subcore_axis_name="s")
_sc_params = pltpu.CompilerParams(needs_layout_passes=False)


def _flags_call(labels):
    @functools.partial(
        pl.kernel,
        mesh=_mesh,
        out_type=jax.ShapeDtypeStruct((NW, K), jnp.int32),
        compiler_params=_sc_params,
        scratch_types=[
            pltpu.VMEM((CT,), jnp.int32),
            pltpu.VMEM((K,), jnp.int32),
            pltpu.VMEM((K,), jnp.int32),
        ]
        + [pltpu.SemaphoreType.DMA] * NCH,
    )
    def k(labels_hbm, flags_hbm, lab_v, flg_v, flg_w, *sems):
        c = lax.axis_index("c")
        s = lax.axis_index("s")
        wid = s * NC + c
        base = wid * C0

        def in_copy(j):
            off, sz = CHUNKS[j]
            return pltpu.make_async_copy(
                labels_hbm.at[pl.ds(base + off, sz)],
                lab_v.at[pl.ds(off, sz)],
                sems[j],
            )

        for j in range(NCH):
            in_copy(j).start()

        zeros = jnp.zeros((L,), jnp.int32)
        for i in range(K // L):
            flg_v[pl.ds(i * L, L)] = zeros
            flg_w[pl.ds(i * L, L)] = zeros

        ones = jnp.ones((L,), jnp.int32)
        for j in range(NCH):
            off, sz = CHUNKS[j]
            in_copy(j).wait()

            def body(i, carry, off=off):
                lv0 = lab_v[pl.ds(off + i * 2 * L, L)]
                lv1 = lab_v[pl.ds(off + i * 2 * L + L, L)]
                plsc.store_scatter(flg_v, [lv0 - 1], ones)
                plsc.store_scatter(flg_w, [lv1 - 1], ones)
                return carry

            lax.fori_loop(0, sz // (2 * L), body, 0, unroll=UNROLL // 2)

        for i in range(K // L):
            flg_v[pl.ds(i * L, L)] = (
                flg_v[pl.ds(i * L, L)] + flg_w[pl.ds(i * L, L)]
            )

        pltpu.sync_copy(flg_v, flags_hbm.at[wid])

    return k(labels)


def _rank_body(flags_ref, peak_row_ref, peak_col_ref, out_ref):
    occ_row = (jnp.sum(flags_ref[...], axis=0, keepdims=True) > 0).astype(
        jnp.float32
    )
    kidx_row = lax.broadcasted_iota(jnp.int32, (1, K), 1)
    kmax = jnp.max(jnp.where(occ_row > 0, kidx_row + 1, 0))

    ii = lax.broadcasted_iota(jnp.int32, (K, K), 0)
    jj = lax.broadcasted_iota(jnp.int32, (K, K), 1)
    iden = (ii == jj).astype(jnp.float32)
    occ_col = lax.dot_general(
        iden,
        occ_row,
        (((1,), (1,)), ((), ())),
        preferred_element_type=jnp.float32,
    )

    inf = jnp.float32(jnp.inf)
    aa_row = jnp.where(
        jnp.logical_and(occ_row == 0.0, kidx_row < kmax), inf, peak_row_ref[...]
    )
    kidx_col = lax.broadcasted_iota(jnp.int32, (K, 1), 0)
    aa_col = jnp.where(
        jnp.logical_and(occ_col == 0.0, kidx_col < kmax), inf, peak_col_ref[...]
    )

    beforeT = jnp.logical_or(
        aa_col < aa_row, jnp.logical_and(aa_col == aa_row, ii < jj)
    )
    rank = jnp.sum(beforeT.astype(jnp.int32), axis=0, keepdims=True)
    out_ref[...] = rank + 1


def _rank_call(flags, peak):
    return pl.pallas_call(
        _rank_body,
        out_shape=jax.ShapeDtypeStruct((1, K), jnp.int32),
    )(flags, peak.reshape(1, K), peak.reshape(K, 1))


def _gather_call(labels, mapping):
    @functools.partial(
        pl.kernel,
        mesh=_mesh,
        out_type=jax.ShapeDtypeStruct((N,), jnp.int32),
        compiler_params=_sc_params,
        scratch_types=[
            pltpu.VMEM((CT,), jnp.int32),
            pltpu.VMEM((CT,), jnp.int32),
            pltpu.VMEM((K,), jnp.int32),
        ]
        + [pltpu.SemaphoreType.DMA] * (2 * NCH + 1),
    )
    def k(labels_hbm, map_hbm, out_hbm, lab_v, out_v, tab_v, *sems):
        c = lax.axis_index("c")
        s = lax.axis_index("s")
        wid = s * NC + c
        base = wid * C0
        sems_in = sems[:NCH]
        sems_out = sems[NCH : 2 * NCH]
        sem_tail = sems[2 * NCH]

        def in_copy(j):
            off, sz = CHUNKS[j]
            return pltpu.make_async_copy(
                labels_hbm.at[pl.ds(base + off, sz)],
                lab_v.at[pl.ds(off, sz)],
                sems_in[j],
            )

        def out_copy(j):
            off, sz = WCHUNKS[j]
            return pltpu.make_async_copy(
                out_v.at[pl.ds(off, sz)],
                out_hbm.at[pl.ds(base + off, sz)],
                sems_out[j],
            )

        def tail_copy():
            return pltpu.make_async_copy(
                out_v.at[pl.ds(C0, TAIL)],
                out_hbm.at[pl.ds(base + C0, TAIL)],
                sem_tail,
            )

        for j in range(NCH):
            in_copy(j).start()
        pltpu.sync_copy(map_hbm, tab_v)

        for j in range(NCH):
            off, sz = CHUNKS[j]
            in_copy(j).wait()

            def body(i, carry, off=off):
                lv = lab_v[pl.ds(off + i * L, L)]
                out_v[pl.ds(off + i * L, L)] = plsc.load_gather(
                    tab_v, [lv - 1]
                )
                return carry

            lax.fori_loop(0, sz // L, body, 0, unroll=UNROLL)
            out_copy(j).start()

        @pl.when(wid == NW - 1)
        def _():
            tail_copy().start()
            tail_copy().wait()

        for j in range(NCH):
            out_copy(j).wait()

    return k(labels, mapping)


def kernel(labels, peak_channel_indices):
    flags = _flags_call(labels)
    mapping = _rank_call(flags, peak_channel_indices)
    return _gather_call(labels, mapping.reshape(K))


# speedup vs baseline: 1.3128x; 1.0417x over previous
import functools

import jax
import jax.numpy as jnp
from jax import lax
from jax.experimental import pallas as pl
from jax.experimental.pallas import tpu as pltpu
from jax.experimental.pallas import tpu_sc as plsc

NC = 2
NS = 16
NW = NC * NS
L = 16

N = 2_000_000
K = 1024

C0 = (N // NW) // L * L
TAIL = N - NW * C0
CT = C0 + TAIL

CH = 8192
_starts = list(range(0, CT, CH))
CHUNKS = [(o, min(CH, CT - o)) for o in _starts]
NCH = len(CHUNKS)
CHG = 16384
_gstarts = list(range(0, CT, CHG))
GCHUNKS = [(o, min(CHG, CT - o)) for o in _gstarts]
NCHG = len(GCHUNKS)
GWCHUNKS = [(o, min(CHG, C0 - o)) for o in _gstarts if o < C0]

UNROLL = 8

_mesh = plsc.VectorSubcoreMesh(core_axis_name="c", subcore_axis_name="s")
_sc_params = pltpu.CompilerParams(needs_layout_passes=False)


def _flags_call(labels):
    @functools.partial(
        pl.kernel,
        mesh=_mesh,
        out_type=jax.ShapeDtypeStruct((NW, K), jnp.int32),
        compiler_params=_sc_params,
        scratch_types=[
            pltpu.VMEM((CT,), jnp.int32),
            pltpu.VMEM((K,), jnp.int32),
            pltpu.VMEM((K,), jnp.int32),
        ]
        + [pltpu.SemaphoreType.DMA] * NCH,
    )
    def k(labels_hbm, flags_hbm, lab_v, flg_v, flg_w, *sems):
        c = lax.axis_index("c")
        s = lax.axis_index("s")
        wid = s * NC + c
        base = wid * C0

        def in_copy(j):
            off, sz = CHUNKS[j]
            return pltpu.make_async_copy(
                labels_hbm.at[pl.ds(base + off, sz)],
                lab_v.at[pl.ds(off, sz)],
                sems[j],
            )

        for j in range(NCH):
            in_copy(j).start()

        zeros = jnp.zeros((L,), jnp.int32)
        for i in range(K // L):
            flg_v[pl.ds(i * L, L)] = zeros
            flg_w[pl.ds(i * L, L)] = zeros

        ones = jnp.ones((L,), jnp.int32)
        for j in range(NCH):
            off, sz = CHUNKS[j]
            in_copy(j).wait()

            def body(i, carry, off=off):
                lv0 = lab_v[pl.ds(off + i * 2 * L, L)]
                lv1 = lab_v[pl.ds(off + i * 2 * L + L, L)]
                plsc.store_scatter(flg_v, [lv0 - 1], ones)
                plsc.store_scatter(flg_w, [lv1 - 1], ones)
                return carry

            lax.fori_loop(0, sz // (2 * L), body, 0, unroll=UNROLL // 2)

        for i in range(K // L):
            flg_v[pl.ds(i * L, L)] = (
                flg_v[pl.ds(i * L, L)] + flg_w[pl.ds(i * L, L)]
            )

        pltpu.sync_copy(flg_v, flags_hbm.at[wid])

    return k(labels)


def _rank_body(flags_ref, peak_row_ref, peak_col_ref, out_ref):
    occ_row = (jnp.sum(flags_ref[...], axis=0, keepdims=True) > 0).astype(
        jnp.float32
    )
    kidx_row = lax.broadcasted_iota(jnp.int32, (1, K), 1)
    kmax = jnp.max(jnp.where(occ_row > 0, kidx_row + 1, 0))

    ii = lax.broadcasted_iota(jnp.int32, (K, K), 0)
    jj = lax.broadcasted_iota(jnp.int32, (K, K), 1)
    iden = (ii == jj).astype(jnp.float32)
    occ_col = lax.dot_general(
        iden,
        occ_row,
        (((1,), (1,)), ((), ())),
        preferred_element_type=jnp.float32,
    )

    inf = jnp.float32(jnp.inf)
    aa_row = jnp.where(
        jnp.logical_and(occ_row == 0.0, kidx_row < kmax), inf, peak_row_ref[...]
    )
    kidx_col = lax.broadcasted_iota(jnp.int32, (K, 1), 0)
    aa_col = jnp.where(
        jnp.logical_and(occ_col == 0.0, kidx_col < kmax), inf, peak_col_ref[...]
    )

    beforeT = jnp.logical_or(
        aa_col < aa_row, jnp.logical_and(aa_col == aa_row, ii < jj)
    )
    rank = jnp.sum(beforeT.astype(jnp.int32), axis=0, keepdims=True)
    out_ref[...] = rank + 1


def _rank_call(flags, peak):
    return pl.pallas_call(
        _rank_body,
        out_shape=jax.ShapeDtypeStruct((1, K), jnp.int32),
    )(flags, peak.reshape(1, K), peak.reshape(K, 1))


def _gather_call(labels, mapping):
    @functools.partial(
        pl.kernel,
        mesh=_mesh,
        out_type=jax.ShapeDtypeStruct((N,), jnp.int32),
        compiler_params=_sc_params,
        scratch_types=[
            pltpu.VMEM((CT,), jnp.int32),
            pltpu.VMEM((K,), jnp.int32),
            pltpu.VMEM((K * L,), jnp.int32),
        ]
        + [pltpu.SemaphoreType.DMA] * (2 * NCHG + 1),
    )
    def k(labels_hbm, map_hbm, out_hbm, lab_v, tab_v, tabr_v, *sems):
        c = lax.axis_index("c")
        s = lax.axis_index("s")
        wid = s * NC + c
        base = wid * C0
        sems_in = sems[:NCHG]
        sems_out = sems[NCHG : 2 * NCHG]
        sem_tail = sems[2 * NCHG]

        def in_copy(j):
            off, sz = GCHUNKS[j]
            return pltpu.make_async_copy(
                labels_hbm.at[pl.ds(base + off, sz)],
                lab_v.at[pl.ds(off, sz)],
                sems_in[j],
            )

        def out_copy(j):
            off, sz = GWCHUNKS[j]
            return pltpu.make_async_copy(
                lab_v.at[pl.ds(off, sz)],
                out_hbm.at[pl.ds(base + off, sz)],
                sems_out[j],
            )

        def tail_copy():
            return pltpu.make_async_copy(
                lab_v.at[pl.ds(C0, TAIL)],
                out_hbm.at[pl.ds(base + C0, TAIL)],
                sem_tail,
            )

        for j in range(NCHG):
            in_copy(j).start()
        pltpu.sync_copy(map_hbm, tab_v)

        zero16 = jnp.zeros((L,), jnp.int32)

        def rep_body(v, carry):
            tabr_v[pl.ds(v * L, L)] = plsc.load_gather(tab_v, [zero16 + v])
            return carry

        lax.fori_loop(0, K, rep_body, 0, unroll=UNROLL)

        lane_off = lax.broadcasted_iota(jnp.int32, (L,), 0) - L

        for j in range(NCHG):
            off, sz = GCHUNKS[j]
            in_copy(j).wait()

            def body(i, carry, off=off):
                lv = lab_v[pl.ds(off + i * L, L)]
                idx = jnp.left_shift(lv, 4) + lane_off
                lab_v[pl.ds(off + i * L, L)] = plsc.load_gather(
                    tabr_v, [idx]
                )
                return carry

            lax.fori_loop(0, sz // L, body, 0, unroll=UNROLL)
            out_copy(j).start()

        @pl.when(wid == NW - 1)
        def _():
            tail_copy().start()
            tail_copy().wait()

        for j in range(NCHG):
            out_copy(j).wait()

    return k(labels, mapping)


def kernel(labels, peak_channel_indices):
    flags = _flags_call(labels)
    mapping = _rank_call(flags, peak_channel_indices)
    return _gather_call(labels, mapping.reshape(K))


# speedup vs baseline: 1.4367x; 1.0944x over previous
import functools

import jax
import jax.numpy as jnp
from jax import lax
from jax.experimental import pallas as pl
from jax.experimental.pallas import tpu as pltpu
from jax.experimental.pallas import tpu_sc as plsc

NC = 2
NS = 16
NW = NC * NS
L = 16

N = 2_000_000
K = 1024

C0 = (N // NW) // L * L
TAIL = N - NW * C0
CT = C0 + TAIL

CH = 8192
_starts = list(range(0, CT, CH))
CHUNKS = [(o, min(CH, CT - o)) for o in _starts]
NCH = len(CHUNKS)
CHG = 16384
_gstarts = list(range(0, CT, CHG))
GCHUNKS = [(o, min(CHG, CT - o)) for o in _gstarts]
NCHG = len(GCHUNKS)
GWCHUNKS = [(o, min(CHG, C0 - o)) for o in _gstarts if o < C0]

UNROLL = 8

_mesh = plsc.VectorSubcoreMesh(core_axis_name="c", subcore_axis_name="s")
_sc_params = pltpu.CompilerParams(needs_layout_passes=False)


def _flags_call(labels):
    @functools.partial(
        pl.kernel,
        mesh=_mesh,
        out_type=jax.ShapeDtypeStruct((NW, K), jnp.int32),
        compiler_params=_sc_params,
        scratch_types=[
            pltpu.VMEM((CT,), jnp.int32),
            pltpu.VMEM((K,), jnp.int32),
            pltpu.VMEM((K,), jnp.int32),
            pltpu.VMEM((K,), jnp.int32),
            pltpu.VMEM((K,), jnp.int32),
        ]
        + [pltpu.SemaphoreType.DMA] * NCH,
    )
    def k(labels_hbm, flags_hbm, lab_v, flg_v, flg_w, flg_x, flg_y, *sems):
        c = lax.axis_index("c")
        s = lax.axis_index("s")
        wid = s * NC + c
        base = wid * C0

        def in_copy(j):
            off, sz = CHUNKS[j]
            return pltpu.make_async_copy(
                labels_hbm.at[pl.ds(base + off, sz)],
                lab_v.at[pl.ds(off, sz)],
                sems[j],
            )

        for j in range(NCH):
            in_copy(j).start()

        zeros = jnp.zeros((L,), jnp.int32)
        for i in range(K // L):
            flg_v[pl.ds(i * L, L)] = zeros
            flg_w[pl.ds(i * L, L)] = zeros
            flg_x[pl.ds(i * L, L)] = zeros
            flg_y[pl.ds(i * L, L)] = zeros

        ones = jnp.ones((L,), jnp.int32)
        for j in range(NCH):
            off, sz = CHUNKS[j]
            in_copy(j).wait()

            def quad(i, carry, off=off):
                lv0 = lab_v[pl.ds(off + i * 4 * L, L)]
                lv1 = lab_v[pl.ds(off + i * 4 * L + L, L)]
                lv2 = lab_v[pl.ds(off + i * 4 * L + 2 * L, L)]
                lv3 = lab_v[pl.ds(off + i * 4 * L + 3 * L, L)]
                plsc.store_scatter(flg_v, [lv0 - 1], ones)
                plsc.store_scatter(flg_w, [lv1 - 1], ones)
                plsc.store_scatter(flg_x, [lv2 - 1], ones)
                plsc.store_scatter(flg_y, [lv3 - 1], ones)
                return carry

            nq = sz // (4 * L)
            lax.fori_loop(0, nq, quad, 0, unroll=2)
            for r in range(nq * 4 * L, sz, 2 * L):
                lv0 = lab_v[pl.ds(off + r, L)]
                lv1 = lab_v[pl.ds(off + r + L, L)]
                plsc.store_scatter(flg_v, [lv0 - 1], ones)
                plsc.store_scatter(flg_w, [lv1 - 1], ones)

        for i in range(K // L):
            flg_v[pl.ds(i * L, L)] = (
                flg_v[pl.ds(i * L, L)] + flg_w[pl.ds(i * L, L)]
            ) + (flg_x[pl.ds(i * L, L)] + flg_y[pl.ds(i * L, L)])

        pltpu.sync_copy(flg_v, flags_hbm.at[wid])

    return k(labels)


def _rank_body(flags_ref, peak_row_ref, peak_col_ref, out_ref):
    occ_row = (jnp.sum(flags_ref[...], axis=0, keepdims=True) > 0).astype(
        jnp.float32
    )
    kidx_row = lax.broadcasted_iota(jnp.int32, (1, K), 1)
    kmax = jnp.max(jnp.where(occ_row > 0, kidx_row + 1, 0))

    ii = lax.broadcasted_iota(jnp.int32, (K, K), 0)
    jj = lax.broadcasted_iota(jnp.int32, (K, K), 1)
    iden = (ii == jj).astype(jnp.float32)
    occ_col = lax.dot_general(
        iden,
        occ_row,
        (((1,), (1,)), ((), ())),
        preferred_element_type=jnp.float32,
    )

    inf = jnp.float32(jnp.inf)
    aa_row = jnp.where(
        jnp.logical_and(occ_row == 0.0, kidx_row < kmax), inf, peak_row_ref[...]
    )
    kidx_col = lax.broadcasted_iota(jnp.int32, (K, 1), 0)
    aa_col = jnp.where(
        jnp.logical_and(occ_col == 0.0, kidx_col < kmax), inf, peak_col_ref[...]
    )

    before = jnp.logical_or(
        aa_row < aa_col, jnp.logical_and(aa_row == aa_col, jj < ii)
    )
    rank = jnp.sum(before.astype(jnp.int32), axis=1, keepdims=True)
    out_ref[...] = jnp.broadcast_to(rank + 1, (K, L))


def _rank_call(flags, peak):
    return pl.pallas_call(
        _rank_body,
        out_shape=jax.ShapeDtypeStruct((K, L), jnp.int32),
    )(flags, peak.reshape(1, K), peak.reshape(K, 1))


def _gather_call(labels, mapping):
    @functools.partial(
        pl.kernel,
        mesh=_mesh,
        out_type=jax.ShapeDtypeStruct((N,), jnp.int32),
        compiler_params=_sc_params,
        scratch_types=[
            pltpu.VMEM((CT,), jnp.int32),
            pltpu.VMEM((K * L,), jnp.int32),
        ]
        + [pltpu.SemaphoreType.DMA] * (2 * NCHG + 1),
    )
    def k(labels_hbm, map_hbm, out_hbm, lab_v, tabr_v, *sems):
        c = lax.axis_index("c")
        s = lax.axis_index("s")
        wid = s * NC + c
        base = wid * C0
        sems_in = sems[:NCHG]
        sems_out = sems[NCHG : 2 * NCHG]
        sem_tail = sems[2 * NCHG]

        def in_copy(j):
            off, sz = GCHUNKS[j]
            return pltpu.make_async_copy(
                labels_hbm.at[pl.ds(base + off, sz)],
                lab_v.at[pl.ds(off, sz)],
                sems_in[j],
            )

        def out_copy(j):
            off, sz = GWCHUNKS[j]
            return pltpu.make_async_copy(
                lab_v.at[pl.ds(off, sz)],
                out_hbm.at[pl.ds(base + off, sz)],
                sems_out[j],
            )

        def tail_copy():
            return pltpu.make_async_copy(
                lab_v.at[pl.ds(C0, TAIL)],
                out_hbm.at[pl.ds(base + C0, TAIL)],
                sem_tail,
            )

        for j in range(NCHG):
            in_copy(j).start()
        pltpu.sync_copy(map_hbm, tabr_v)

        lane_off = lax.broadcasted_iota(jnp.int32, (L,), 0) - L

        for j in range(NCHG):
            off, sz = GCHUNKS[j]
            in_copy(j).wait()

            def body(i, carry, off=off):
                lv = lab_v[pl.ds(off + i * L, L)]
                idx = jnp.left_shift(lv, 4) + lane_off
                lab_v[pl.ds(off + i * L, L)] = plsc.load_gather(
                    tabr_v, [idx]
                )
                return carry

            lax.fori_loop(0, sz // L, body, 0, unroll=UNROLL)
            out_copy(j).start()

        @pl.when(wid == NW - 1)
        def _():
            tail_copy().start()
            tail_copy().wait()

        for j in range(NCHG):
            out_copy(j).wait()

    return k(labels, mapping)


def kernel(labels, peak_channel_indices):
    flags = _flags_call(labels)
    mapping = _rank_call(flags, peak_channel_indices)
    return _gather_call(labels, mapping.reshape(K * L))
